# baseline (speedup 1.0000x reference)
import functools

import jax
import jax.numpy as jnp
from jax import lax
from jax.experimental import pallas as pl
from jax.experimental.pallas import tpu as pltpu
from jax.experimental.pallas import tpu_sc as plsc

N = 16777216
NUM_BINS = 15
NC = 2
NS = 16
NW = NC * NS
LANES = 16
PER_W = N // NW
CHUNK = 16384
NCHUNK = PER_W // CHUNK
VREGS = CHUNK // LANES
PHASES = 8


def _sc_body(pred_hbm, conf_hbm, targ_hbm, out_hbm,
             conf0, conf1, pred0, pred1, targ0, targ1,
             acc_cc, acc_cnf, res_v,
             sem0, sem1):
    wid = lax.axis_index("s") * NC + lax.axis_index("c")
    base = wid * PER_W
    sems = (sem0, sem1)
    confs = (conf0, conf1)
    preds = (pred0, pred1)
    targs = (targ0, targ1)

    lane = lax.iota(jnp.int32, LANES)
    lane_j = [lane + j * (LANES * LANES) for j in range(PHASES)]
    zeros = jnp.zeros((LANES,), jnp.float32)
    zeros_i = jnp.zeros((LANES,), jnp.int32)

    for l in range(PHASES * LANES):
        acc_cc[pl.ds(l * LANES, LANES)] = zeros_i
        acc_cnf[pl.ds(l * LANES, LANES)] = zeros

    def start_chunk(k, slot):
        off = base + k * CHUNK
        pltpu.async_copy(conf_hbm.at[pl.ds(off, CHUNK)], confs[slot], sems[slot])
        pltpu.async_copy(pred_hbm.at[pl.ds(off, CHUNK)], preds[slot], sems[slot])
        pltpu.async_copy(targ_hbm.at[pl.ds(off, CHUNK)], targs[slot], sems[slot])

    def wait_chunk(k, slot):
        off = base + k * CHUNK
        pltpu.make_async_copy(conf_hbm.at[pl.ds(off, CHUNK)], confs[slot], sems[slot]).wait()
        pltpu.make_async_copy(pred_hbm.at[pl.ds(off, CHUNK)], preds[slot], sems[slot]).wait()
        pltpu.make_async_copy(targ_hbm.at[pl.ds(off, CHUNK)], targs[slot], sems[slot]).wait()

    def compute_chunk(slot):
        conf_r = confs[slot]
        pred_r = preds[slot]
        targ_r = targs[slot]

        @plsc.parallel_loop(0, VREGS, step=PHASES, unroll=16)
        def _inner(i):
            for j in range(PHASES):
                off = (i + j) * LANES
                conf = conf_r[pl.ds(off, LANES)]
                pred = pred_r[pl.ds(off, LANES)]
                targ = targ_r[pl.ds(off, LANES)]
                t = (conf * jnp.float32(NUM_BINS * LANES)).astype(jnp.int32)
                idx = (t & -LANES) | lane_j[j]
                cc = jnp.where(pred == targ, jnp.int32(65537), jnp.int32(65536))
                plsc.addupdate_scatter(acc_cc, [idx], cc)
                plsc.addupdate_scatter(acc_cnf, [idx], conf)

    start_chunk(0, 0)

    @pl.loop(0, NCHUNK // 2)
    def _outer(kk):
        for s in (0, 1):
            k = kk * 2 + s

            @pl.when(k + 1 < NCHUNK)
            def _():
                start_chunk(k + 1, 1 - s)

            wait_chunk(k, s)
            compute_chunk(s)

    TB = LANES * LANES
    for v in range(LANES):
        cc_tot = zeros_i
        cnf_tot = zeros
        for j in range(PHASES):
            cc_tot = cc_tot + acc_cc[pl.ds(j * TB + v * LANES, LANES)]
            cnf_tot = cnf_tot + acc_cnf[pl.ds(j * TB + v * LANES, LANES)]
        res_v[0, v, :] = (cc_tot >> 16).astype(jnp.float32)
        res_v[1, v, :] = (cc_tot & 0xFFFF).astype(jnp.float32)
        res_v[2, v, :] = cnf_tot
    pltpu.sync_copy(res_v, out_hbm.at[wid])


_TB = LANES * LANES
_sc_hist = functools.partial(
    pl.kernel,
    out_type=jax.ShapeDtypeStruct((NW, 3, LANES, LANES), jnp.float32),
    mesh=plsc.VectorSubcoreMesh(core_axis_name="c", subcore_axis_name="s"),
    compiler_params=pltpu.CompilerParams(needs_layout_passes=False),
    scratch_types=[
        pltpu.VMEM((CHUNK,), jnp.float32),
        pltpu.VMEM((CHUNK,), jnp.float32),
        pltpu.VMEM((CHUNK,), jnp.int32),
        pltpu.VMEM((CHUNK,), jnp.int32),
        pltpu.VMEM((CHUNK,), jnp.int32),
        pltpu.VMEM((CHUNK,), jnp.int32),
        pltpu.VMEM((PHASES * _TB,), jnp.int32),
        pltpu.VMEM((PHASES * _TB,), jnp.float32),
        pltpu.VMEM((3, LANES, LANES), jnp.float32),
        pltpu.SemaphoreType.DMA,
        pltpu.SemaphoreType.DMA,
    ],
)(_sc_body)


def _combine_body(p_ref, o_ref):
    p = p_ref[...]
    cnt = jnp.sum(p[:, 0, :, :], axis=(0, 2))
    cor = jnp.sum(p[:, 1, :, :], axis=(0, 2))
    cnf = jnp.sum(p[:, 2, :, :], axis=(0, 2))
    safe = jnp.maximum(cnt, 1.0)
    contrib = (cnt / jnp.float32(N)) * jnp.abs(cor / safe - cnf / safe)
    valid = (jnp.arange(LANES) < NUM_BINS) & (cnt > 0)
    ece = jnp.sum(jnp.where(valid, contrib, 0.0))
    o_ref[0, 0] = ece


def _combine(partials):
    return pl.pallas_call(
        _combine_body,
        out_shape=jax.ShapeDtypeStruct((1, 1), jnp.float32),
        out_specs=pl.BlockSpec(memory_space=pltpu.SMEM),
    )(partials)


def kernel(predictions, confidences, targets):
    partials = _sc_hist(predictions, confidences, targets)
    ece = _combine(partials)
    return ece[0, 0]


# speedup vs baseline: 1.2039x; 1.2039x over previous
import functools

import jax
import jax.numpy as jnp
from jax import lax
from jax.experimental import pallas as pl
from jax.experimental.pallas import tpu as pltpu
from jax.experimental.pallas import tpu_sc as plsc

N = 16777216
NUM_BINS = 15
NC = 2
NS = 16
NW = NC * NS
LANES = 16
PER_W = N // NW
CHUNK = 8192
NCHUNK = PER_W // CHUNK
VREGS = CHUNK // LANES
PHASES = 8


def _sc_body(pred_hbm, conf_hbm, targ_hbm, out_hbm,
             conf0, conf1, pred0, pred1, targ0, targ1,
             acc_cc, acc_cnf, res_v,
             sem0, sem1):
    wid = lax.axis_index("s") * NC + lax.axis_index("c")
    base = wid * PER_W
    sems = (sem0, sem1)
    confs = (conf0, conf1)
    preds = (pred0, pred1)
    targs = (targ0, targ1)

    lane = lax.iota(jnp.int32, LANES)
    lane_j = [lane + j * (LANES * LANES) for j in range(PHASES)]
    zeros = jnp.zeros((LANES,), jnp.float32)
    zeros_i = jnp.zeros((LANES,), jnp.int32)

    for l in range(PHASES * LANES):
        acc_cc[pl.ds(l * LANES, LANES)] = zeros_i
        acc_cnf[pl.ds(l * LANES, LANES)] = zeros

    def start_chunk(k, slot):
        off = base + k * CHUNK
        pltpu.async_copy(conf_hbm.at[pl.ds(off, CHUNK)], confs[slot], sems[slot])
        pltpu.async_copy(pred_hbm.at[pl.ds(off, CHUNK)], preds[slot], sems[slot])
        pltpu.async_copy(targ_hbm.at[pl.ds(off, CHUNK)], targs[slot], sems[slot])

    def wait_chunk(k, slot):
        off = base + k * CHUNK
        pltpu.make_async_copy(conf_hbm.at[pl.ds(off, CHUNK)], confs[slot], sems[slot]).wait()
        pltpu.make_async_copy(pred_hbm.at[pl.ds(off, CHUNK)], preds[slot], sems[slot]).wait()
        pltpu.make_async_copy(targ_hbm.at[pl.ds(off, CHUNK)], targs[slot], sems[slot]).wait()

    def compute_chunk(slot):
        conf_r = confs[slot]
        pred_r = preds[slot]
        targ_r = targs[slot]

        @plsc.parallel_loop(0, VREGS, step=PHASES, unroll=8)
        def _inner(i):
            for j in range(PHASES):
                off = (i + j) * LANES
                conf = conf_r[pl.ds(off, LANES)]
                pred = pred_r[pl.ds(off, LANES)]
                targ = targ_r[pl.ds(off, LANES)]
                t = (conf * jnp.float32(NUM_BINS * LANES)).astype(jnp.int32)
                idx = (t & -LANES) | lane_j[j]
                cc = jnp.where(pred == targ, jnp.int32(65537), jnp.int32(65536))
                plsc.addupdate_scatter(acc_cc, [idx], cc)
                plsc.addupdate_scatter(acc_cnf, [idx], conf)

    start_chunk(0, 0)

    @pl.loop(0, NCHUNK // 2)
    def _outer(kk):
        for s in (0, 1):
            k = kk * 2 + s

            @pl.when(k + 1 < NCHUNK)
            def _():
                start_chunk(k + 1, 1 - s)

            wait_chunk(k, s)
            compute_chunk(s)

    TB = LANES * LANES
    for v in range(LANES):
        cc_tot = zeros_i
        cnf_tot = zeros
        for j in range(PHASES):
            cc_tot = cc_tot + acc_cc[pl.ds(j * TB + v * LANES, LANES)]
            cnf_tot = cnf_tot + acc_cnf[pl.ds(j * TB + v * LANES, LANES)]
        res_v[0, v, :] = (cc_tot >> 16).astype(jnp.float32)
        res_v[1, v, :] = (cc_tot & 0xFFFF).astype(jnp.float32)
        res_v[2, v, :] = cnf_tot
    pltpu.sync_copy(res_v, out_hbm.at[wid])


_TB = LANES * LANES
_sc_hist = functools.partial(
    pl.kernel,
    out_type=jax.ShapeDtypeStruct((NW, 3, LANES, LANES), jnp.float32),
    mesh=plsc.VectorSubcoreMesh(core_axis_name="c", subcore_axis_name="s"),
    compiler_params=pltpu.CompilerParams(needs_layout_passes=False),
    scratch_types=[
        pltpu.VMEM((CHUNK,), jnp.float32),
        pltpu.VMEM((CHUNK,), jnp.float32),
        pltpu.VMEM((CHUNK,), jnp.int32),
        pltpu.VMEM((CHUNK,), jnp.int32),
        pltpu.VMEM((CHUNK,), jnp.int32),
        pltpu.VMEM((CHUNK,), jnp.int32),
        pltpu.VMEM((PHASES * _TB,), jnp.int32),
        pltpu.VMEM((PHASES * _TB,), jnp.float32),
        pltpu.VMEM((3, LANES, LANES), jnp.float32),
        pltpu.SemaphoreType.DMA,
        pltpu.SemaphoreType.DMA,
    ],
)(_sc_body)


def _combine_body(p_ref, o_ref):
    p = p_ref[...]
    cnt = jnp.sum(p[:, 0, :, :], axis=(0, 2))
    cor = jnp.sum(p[:, 1, :, :], axis=(0, 2))
    cnf = jnp.sum(p[:, 2, :, :], axis=(0, 2))
    safe = jnp.maximum(cnt, 1.0)
    contrib = (cnt / jnp.float32(N)) * jnp.abs(cor / safe - cnf / safe)
    valid = (jnp.arange(LANES) < NUM_BINS) & (cnt > 0)
    ece = jnp.sum(jnp.where(valid, contrib, 0.0))
    o_ref[0, 0] = ece


def _combine(partials):
    return pl.pallas_call(
        _combine_body,
        out_shape=jax.ShapeDtypeStruct((1, 1), jnp.float32),
        out_specs=pl.BlockSpec(memory_space=pltpu.SMEM),
    )(partials)


def kernel(predictions, confidences, targets):
    partials = _sc_hist(predictions, confidences, targets)
    ece = _combine(partials)
    return ece[0, 0]


# speedup vs baseline: 1.2272x; 1.0194x over previous
import functools

import jax
import jax.numpy as jnp
from jax import lax
from jax.experimental import pallas as pl
from jax.experimental.pallas import tpu as pltpu
from jax.experimental.pallas import tpu_sc as plsc

N = 16777216
NUM_BINS = 15
NC = 2
NS = 16
NW = NC * NS
LANES = 16
PER_W = N // NW
CHUNK = 16384
NCHUNK = PER_W // CHUNK
VREGS = CHUNK // LANES
PHASES = 8


def _sc_body(pred_hbm, conf_hbm, targ_hbm, out_hbm,
             conf0, conf1, pred0, pred1, targ0, targ1,
             acc_cc, acc_cnf, res_v,
             sem0, sem1):
    wid = lax.axis_index("s") * NC + lax.axis_index("c")
    base = wid * PER_W
    sems = (sem0, sem1)
    confs = (conf0, conf1)
    preds = (pred0, pred1)
    targs = (targ0, targ1)

    lane = lax.iota(jnp.int32, LANES)
    lane_j = [lane + j * (LANES * LANES) for j in range(PHASES)]
    zeros = jnp.zeros((LANES,), jnp.float32)
    zeros_i = jnp.zeros((LANES,), jnp.int32)

    for l in range(PHASES * LANES):
        acc_cc[pl.ds(l * LANES, LANES)] = zeros_i
        acc_cnf[pl.ds(l * LANES, LANES)] = zeros

    def start_chunk(k, slot):
        off = base + k * CHUNK
        pltpu.async_copy(conf_hbm.at[pl.ds(off, CHUNK)], confs[slot], sems[slot])
        pltpu.async_copy(pred_hbm.at[pl.ds(off, CHUNK)], preds[slot], sems[slot])
        pltpu.async_copy(targ_hbm.at[pl.ds(off, CHUNK)], targs[slot], sems[slot])

    def wait_chunk(k, slot):
        off = base + k * CHUNK
        pltpu.make_async_copy(conf_hbm.at[pl.ds(off, CHUNK)], confs[slot], sems[slot]).wait()
        pltpu.make_async_copy(pred_hbm.at[pl.ds(off, CHUNK)], preds[slot], sems[slot]).wait()
        pltpu.make_async_copy(targ_hbm.at[pl.ds(off, CHUNK)], targs[slot], sems[slot]).wait()

    def compute_chunk(slot):
        conf_r = confs[slot]
        pred_r = preds[slot]
        targ_r = targs[slot]

        @plsc.parallel_loop(0, VREGS, step=PHASES, unroll=8)
        def _inner(i):
            for j in range(PHASES):
                off = (i + j) * LANES
                conf = conf_r[pl.ds(off, LANES)]
                pred = pred_r[pl.ds(off, LANES)]
                targ = targ_r[pl.ds(off, LANES)]
                t = (conf * jnp.float32(NUM_BINS * LANES)).astype(jnp.int32)
                idx = (t & -LANES) | lane_j[j]
                cc = jnp.where(pred == targ, jnp.int32(65537), jnp.int32(65536))
                plsc.addupdate_scatter(acc_cc, [idx], cc)
                plsc.addupdate_scatter(acc_cnf, [idx], conf)

    start_chunk(0, 0)

    @pl.loop(0, NCHUNK // 2)
    def _outer(kk):
        for s in (0, 1):
            k = kk * 2 + s

            @pl.when(k + 1 < NCHUNK)
            def _():
                start_chunk(k + 1, 1 - s)

            wait_chunk(k, s)
            compute_chunk(s)

    TB = LANES * LANES
    for v in range(LANES):
        cc_tot = zeros_i
        cnf_tot = zeros
        for j in range(PHASES):
            cc_tot = cc_tot + acc_cc[pl.ds(j * TB + v * LANES, LANES)]
            cnf_tot = cnf_tot + acc_cnf[pl.ds(j * TB + v * LANES, LANES)]
        res_v[0, v, :] = (cc_tot >> 16).astype(jnp.float32)
        res_v[1, v, :] = (cc_tot & 0xFFFF).astype(jnp.float32)
        res_v[2, v, :] = cnf_tot
    pltpu.sync_copy(res_v, out_hbm.at[wid])


_TB = LANES * LANES
_sc_hist = functools.partial(
    pl.kernel,
    out_type=jax.ShapeDtypeStruct((NW, 3, LANES, LANES), jnp.float32),
    mesh=plsc.VectorSubcoreMesh(core_axis_name="c", subcore_axis_name="s"),
    compiler_params=pltpu.CompilerParams(needs_layout_passes=False),
    scratch_types=[
        pltpu.VMEM((CHUNK,), jnp.float32),
        pltpu.VMEM((CHUNK,), jnp.float32),
        pltpu.VMEM((CHUNK,), jnp.int32),
        pltpu.VMEM((CHUNK,), jnp.int32),
        pltpu.VMEM((CHUNK,), jnp.int32),
        pltpu.VMEM((CHUNK,), jnp.int32),
        pltpu.VMEM((PHASES * _TB,), jnp.int32),
        pltpu.VMEM((PHASES * _TB,), jnp.float32),
        pltpu.VMEM((3, LANES, LANES), jnp.float32),
        pltpu.SemaphoreType.DMA,
        pltpu.SemaphoreType.DMA,
    ],
)(_sc_body)


def _combine_body(p_ref, o_ref):
    p = p_ref[...]
    cnt = jnp.sum(p[:, 0, :, :], axis=(0, 2))
    cor = jnp.sum(p[:, 1, :, :], axis=(0, 2))
    cnf = jnp.sum(p[:, 2, :, :], axis=(0, 2))
    safe = jnp.maximum(cnt, 1.0)
    contrib = (cnt / jnp.float32(N)) * jnp.abs(cor / safe - cnf / safe)
    valid = (jnp.arange(LANES) < NUM_BINS) & (cnt > 0)
    ece = jnp.sum(jnp.where(valid, contrib, 0.0))
    o_ref[0, 0] = ece


def _combine(partials):
    return pl.pallas_call(
        _combine_body,
        out_shape=jax.ShapeDtypeStruct((1, 1), jnp.float32),
        out_specs=pl.BlockSpec(memory_space=pltpu.SMEM),
    )(partials)


def kernel(predictions, confidences, targets):
    partials = _sc_hist(predictions, confidences, targets)
    ece = _combine(partials)
    return ece[0, 0]


# speedup vs baseline: 1.2285x; 1.0011x over previous
import functools

import jax
import jax.numpy as jnp
from jax import lax
from jax.experimental import pallas as pl
from jax.experimental.pallas import tpu as pltpu
from jax.experimental.pallas import tpu_sc as plsc

N = 16777216
NUM_BINS = 15
NC = 2
NS = 16
NW = NC * NS
LANES = 16
PER_W = N // NW
CHUNK = 16384
NCHUNK = PER_W // CHUNK
VREGS = CHUNK // LANES
PHASES = 8


def _sc_body(pred_hbm, conf_hbm, targ_hbm, out_hbm,
             conf0, conf1, pred0, pred1, targ0, targ1,
             acc_cc, acc_cnf, res_v,
             sem0, sem1):
    wid = lax.axis_index("s") * NC + lax.axis_index("c")
    base = wid * PER_W
    sems = (sem0, sem1)
    confs = (conf0, conf1)
    preds = (pred0, pred1)
    targs = (targ0, targ1)

    lane = lax.iota(jnp.int32, LANES)
    lane_j = [lane + j * (LANES * LANES) for j in range(PHASES)]
    zeros = jnp.zeros((LANES,), jnp.float32)
    zeros_i = jnp.zeros((LANES,), jnp.int32)

    for l in range(PHASES * LANES):
        acc_cc[pl.ds(l * LANES, LANES)] = zeros_i
        acc_cnf[pl.ds(l * LANES, LANES)] = zeros

    def start_chunk(k, slot):
        off = base + k * CHUNK
        pltpu.async_copy(conf_hbm.at[pl.ds(off, CHUNK)], confs[slot], sems[slot])
        pltpu.async_copy(pred_hbm.at[pl.ds(off, CHUNK)], preds[slot], sems[slot])
        pltpu.async_copy(targ_hbm.at[pl.ds(off, CHUNK)], targs[slot], sems[slot])

    def wait_chunk(k, slot):
        off = base + k * CHUNK
        pltpu.make_async_copy(conf_hbm.at[pl.ds(off, CHUNK)], confs[slot], sems[slot]).wait()
        pltpu.make_async_copy(pred_hbm.at[pl.ds(off, CHUNK)], preds[slot], sems[slot]).wait()
        pltpu.make_async_copy(targ_hbm.at[pl.ds(off, CHUNK)], targs[slot], sems[slot]).wait()

    def compute_chunk(slot):
        conf_r = confs[slot]
        pred_r = preds[slot]
        targ_r = targs[slot]

        @plsc.parallel_loop(0, VREGS, step=PHASES, unroll=8)
        def _inner(i):
            for j in range(PHASES):
                off = (i + j) * LANES
                conf = conf_r[pl.ds(off, LANES)]
                pred = pred_r[pl.ds(off, LANES)]
                targ = targ_r[pl.ds(off, LANES)]
                t = (conf * jnp.float32(NUM_BINS * LANES)).astype(jnp.int32)
                idx = (t & -LANES) | lane_j[j]
                cc = jnp.where(pred == targ, jnp.int32(65537), jnp.int32(65536))
                plsc.addupdate_scatter(acc_cc, [idx], cc)
                plsc.addupdate_scatter(acc_cnf, [idx], conf)

    start_chunk(0, 0)

    @pl.loop(0, NCHUNK // 2)
    def _outer(kk):
        for s in (0, 1):
            k = kk * 2 + s

            @pl.when(k + 1 < NCHUNK)
            def _():
                start_chunk(k + 1, 1 - s)

            wait_chunk(k, s)
            compute_chunk(s)

    TB = LANES * LANES
    for v in range(LANES):
        cc_tot = zeros_i
        cnf_tot = zeros
        for j in range(PHASES):
            cc_tot = cc_tot + acc_cc[pl.ds(j * TB + v * LANES, LANES)]
            cnf_tot = cnf_tot + acc_cnf[pl.ds(j * TB + v * LANES, LANES)]
        res_v[0, v, :] = (cc_tot >> 16).astype(jnp.float32)
        res_v[1, v, :] = (cc_tot & 0xFFFF).astype(jnp.float32)
        res_v[2, v, :] = cnf_tot
    pltpu.sync_copy(res_v, out_hbm.at[wid])


_TB = LANES * LANES
_sc_hist = functools.partial(
    pl.kernel,
    out_type=jax.ShapeDtypeStruct((NW, 3, LANES, LANES), jnp.float32),
    mesh=plsc.VectorSubcoreMesh(core_axis_name="c", subcore_axis_name="s", num_cores=NC, num_subcores=NS),
    compiler_params=pltpu.CompilerParams(needs_layout_passes=False),
    scratch_types=[
        pltpu.VMEM((CHUNK,), jnp.float32),
        pltpu.VMEM((CHUNK,), jnp.float32),
        pltpu.VMEM((CHUNK,), jnp.int32),
        pltpu.VMEM((CHUNK,), jnp.int32),
        pltpu.VMEM((CHUNK,), jnp.int32),
        pltpu.VMEM((CHUNK,), jnp.int32),
        pltpu.VMEM((PHASES * _TB,), jnp.int32),
        pltpu.VMEM((PHASES * _TB,), jnp.float32),
        pltpu.VMEM((3, LANES, LANES), jnp.float32),
        pltpu.SemaphoreType.DMA,
        pltpu.SemaphoreType.DMA,
    ],
)(_sc_body)


def _combine_body(p_ref, o_ref):
    p = p_ref[...]
    cnt = jnp.sum(p[:, 0, :, :], axis=(0, 2))
    cor = jnp.sum(p[:, 1, :, :], axis=(0, 2))
    cnf = jnp.sum(p[:, 2, :, :], axis=(0, 2))
    safe = jnp.maximum(cnt, 1.0)
    contrib = (cnt / jnp.float32(N)) * jnp.abs(cor / safe - cnf / safe)
    valid = (jnp.arange(LANES) < NUM_BINS) & (cnt > 0)
    ece = jnp.sum(jnp.where(valid, contrib, 0.0))
    o_ref[0, 0] = ece


def _combine(partials):
    return pl.pallas_call(
        _combine_body,
        out_shape=jax.ShapeDtypeStruct((1, 1), jnp.float32),
        out_specs=pl.BlockSpec(memory_space=pltpu.SMEM),
    )(partials)


def kernel(predictions, confidences, targets):
    partials = _sc_hist(predictions, confidences, targets)
    ece = _combine(partials)
    return ece[0, 0]
